# Initial kernel scaffold; baseline (speedup 1.0000x reference)
import jax
import jax.numpy as jnp
from jax.experimental import pallas as pl


def kernel(x, W1l, b1l, W1r, W2l, b2l, W2r, Wa, ba, Wb, bb, edge_index, pos_edge_index, neg_edge_index):
    raise NotImplementedError("write your pallas kernel here")


# speedup vs baseline: 3.9205x; 3.9205x over previous
import functools

import jax
import jax.numpy as jnp
from jax import lax
from jax.experimental import pallas as pl
from jax.experimental.pallas import tpu as pltpu
from jax.experimental.pallas import tpu_sc as plsc

N = 10000
NPAD = 10240
D = 128
DH = D // 2
E = 320000
PE = 100000
PE2 = 2 * PE
PEPAD = 204800

NC = 2
NS = 16
NW = NC * NS
RPT = NPAD // NS

ET = E // NS
AB = 400
ASUB = 4
ABS = AB // ASUB
ANB = ET // AB

DW = PEPAD // NW
DB = 400
DNB = DW // DB

_f32 = jnp.float32
_i32 = jnp.int32


def _sc_mesh():
    return plsc.VectorSubcoreMesh(
        core_axis_name="c", subcore_axis_name="s", num_cores=NC, num_subcores=NS
    )


def _agg_body(with_deg, valsL, valsR, src, dst2, *rest):
    if with_deg:
        (outL, outR, degout, srcb, dstb, rows, ones, zbuf, acc, degsh,
         sem) = rest
    else:
        (outL, outR, srcb, dstb, rows, zbuf, acc, sem) = rest
    cid = lax.axis_index("c")
    sid = lax.axis_index("s")
    z16 = jnp.zeros((16,), _f32)

    def _zrow(r, c):
        for k in range(DH // 16):
            rows[r, pl.ds(k * 16, 16)] = z16
        return c

    lax.fori_loop(0, AB, _zrow, 0)

    def _zb(k, c):
        zbuf[pl.ds(k * 16, 16)] = z16
        return c

    lax.fori_loop(0, RPT // 16, _zb, 0)

    r0 = sid * RPT
    half = RPT // 2
    pltpu.sync_copy(rows.at[pl.ds(0, half)], acc.at[pl.ds(r0, half)])
    pltpu.sync_copy(rows.at[pl.ds(0, half)], acc.at[pl.ds(r0 + half, half)])
    if with_deg:
        o16 = jnp.ones((16,), _f32)
        for k in range(112 // 16):
            ones[pl.ds(k * 16, 16)] = o16
        pltpu.sync_copy(zbuf, degsh.at[pl.ds(r0, RPT)])
    plsc.subcore_barrier()

    base = sid * ET
    rbase = sid * (ET // ABS)

    def _step(j, c):
        pltpu.sync_copy(src.at[pl.ds(base + j * AB, AB)], srcb)
        pltpu.sync_copy(dst2.at[pl.ds(rbase + j * ASUB, ASUB)], dstb)

        @pl.when(cid == 0)
        def _():
            pltpu.async_copy(valsL.at[srcb], rows, sem).wait()

        @pl.when(cid == 1)
        def _():
            pltpu.async_copy(valsR.at[srcb], rows, sem).wait()

        for t in range(ASUB):
            pltpu.sync_copy(rows.at[pl.ds(t * ABS, ABS)],
                            acc.at[dstb.at[t]], add=True)
        if with_deg:
            @pl.when(cid == 0)
            def _():
                for t in range(ASUB):
                    pltpu.sync_copy(ones.at[pl.ds(0, ABS)],
                                    degsh.at[dstb.at[t]], add=True)
        return c

    lax.fori_loop(0, ANB, _step, 0)
    plsc.subcore_barrier()

    @pl.when(cid == 0)
    def _():
        pltpu.sync_copy(acc.at[pl.ds(r0, RPT)], outL.at[pl.ds(r0, RPT)])

    @pl.when(cid == 1)
    def _():
        pltpu.sync_copy(acc.at[pl.ds(r0, RPT)], outR.at[pl.ds(r0, RPT)])

    if with_deg:
        @pl.when(cid == 0)
        def _():
            pltpu.sync_copy(degsh.at[pl.ds(r0, RPT)], degout.at[pl.ds(r0, RPT)])


_sc_params = pltpu.CompilerParams(use_tc_tiling_on_sc=False)

_agg_deg_call = pl.kernel(
    functools.partial(_agg_body, True),
    out_type=[
        jax.ShapeDtypeStruct((NPAD, DH), _f32),
        jax.ShapeDtypeStruct((NPAD, DH), _f32),
        jax.ShapeDtypeStruct((NPAD,), _f32),
    ],
    mesh=_sc_mesh(),
    compiler_params=_sc_params,
    scratch_types=[
        pltpu.VMEM((AB,), _i32),
        pltpu.VMEM((ASUB, ABS), _i32),
        pltpu.VMEM((AB, DH), _f32),
        pltpu.VMEM((112,), _f32),
        pltpu.VMEM((RPT,), _f32),
        pltpu.VMEM_SHARED((NPAD, DH), _f32),
        pltpu.VMEM_SHARED((NPAD,), _f32),
        pltpu.SemaphoreType.DMA,
    ],
)

_agg_call = pl.kernel(
    functools.partial(_agg_body, False),
    out_type=[
        jax.ShapeDtypeStruct((NPAD, DH), _f32),
        jax.ShapeDtypeStruct((NPAD, DH), _f32),
    ],
    mesh=_sc_mesh(),
    compiler_params=_sc_params,
    scratch_types=[
        pltpu.VMEM((AB,), _i32),
        pltpu.VMEM((ASUB, ABS), _i32),
        pltpu.VMEM((AB, DH), _f32),
        pltpu.VMEM((RPT,), _f32),
        pltpu.VMEM_SHARED((NPAD, DH), _f32),
        pltpu.SemaphoreType.DMA,
    ],
)


def _dec_body(p, q, srce, dste, rout, srcb, dstb, bufa, bufb, sema, semb):
    cid = lax.axis_index("c")
    sid = lax.axis_index("s")
    wid = cid * NS + sid
    base = wid * DW

    def _step(j, c):
        off = base + j * DB
        pltpu.sync_copy(srce.at[pl.ds(off, DB)], srcb)
        pltpu.sync_copy(dste.at[pl.ds(off, DB)], dstb)
        ca = pltpu.async_copy(p.at[srcb], bufa, sema)
        cb = pltpu.async_copy(q.at[dstb], bufb, semb)
        ca.wait()
        cb.wait()

        def _addrow(r, cc):
            for k in range(D // 16):
                s = pl.ds(k * 16, 16)
                bufa[r, s] = bufa[r, s] + bufb[r, s]
            return cc

        lax.fori_loop(0, DB, _addrow, 0)
        pltpu.sync_copy(bufa, rout.at[pl.ds(off, DB)])
        return c

    lax.fori_loop(0, DNB, _step, 0)


_dec_call = pl.kernel(
    _dec_body,
    out_type=jax.ShapeDtypeStruct((PEPAD, D), _f32),
    mesh=_sc_mesh(),
    compiler_params=_sc_params,
    scratch_types=[
        pltpu.VMEM((DB,), _i32),
        pltpu.VMEM((DB,), _i32),
        pltpu.VMEM((DB, D), _f32),
        pltpu.VMEM((DB, D), _f32),
        pltpu.SemaphoreType.DMA,
        pltpu.SemaphoreType.DMA,
    ],
)


BM = 512


def _hdot(a, w):
    return jnp.dot(a, w, preferred_element_type=_f32,
                   precision=lax.Precision.HIGHEST)


def _lin1_body(al, ar, d, xr, wlt, wlb, bl, wr, o):
    inv = 1.0 / jnp.maximum(d[...], 1.0)
    z = (_hdot(al[...] * inv, wlt[...]) + _hdot(ar[...] * inv, wlb[...])
         + _hdot(xr[...], wr[...]) + bl[...])
    o[...] = jnp.maximum(z, 0.0)


def _lin2_body(al, ar, d, zr, wlt, wlb, bl, wr, wal, bar, war, p, q):
    inv = 1.0 / jnp.maximum(d[...], 1.0)
    z = (_hdot(al[...] * inv, wlt[...]) + _hdot(ar[...] * inv, wlb[...])
         + _hdot(zr[...], wr[...]) + bl[...])
    p[...] = _hdot(z, wal[...]) + bar[...]
    q[...] = _hdot(z, war[...])


BM3 = 2048


def _dec2_body(r, wb, bb, o):
    h = jnp.maximum(r[...], 0.0)
    o[...] = jnp.sum(h * wb[...], axis=1, keepdims=True) + bb[...]


def _row_spec(bm):
    return pl.BlockSpec((bm, D), lambda i: (i, 0))


def _full_spec(shape):
    return pl.BlockSpec(shape, lambda i: tuple(0 for _ in shape))


def kernel(x, W1l, b1l, W1r, W2l, b2l, W2r, Wa, ba, Wb, bb,
           edge_index, pos_edge_index, neg_edge_index):
    xpad = jnp.pad(x, ((0, NPAD - N), (0, 0)))
    src = edge_index[0].astype(_i32)
    dst2 = edge_index[1].astype(_i32).reshape(E // ABS, ABS)

    pad = jnp.zeros((PEPAD - PE2,), _i32)
    se = jnp.concatenate([pos_edge_index[0].astype(_i32),
                          neg_edge_index[0].astype(_i32), pad])
    de = jnp.concatenate([pos_edge_index[1].astype(_i32),
                          neg_edge_index[1].astype(_i32), pad])

    a1L, a1R, deg = _agg_deg_call(xpad[:, :DH], xpad[:, DH:], src, dst2)
    d2 = deg.reshape(NPAD, 1)

    W1lT = W1l.T
    W2lT = W2l.T
    z1 = pl.pallas_call(
        _lin1_body,
        grid=(NPAD // BM,),
        in_specs=[
            pl.BlockSpec((BM, DH), lambda i: (i, 0)),
            pl.BlockSpec((BM, DH), lambda i: (i, 0)),
            pl.BlockSpec((BM, 1), lambda i: (i, 0)),
            _row_spec(BM),
            _full_spec((DH, D)), _full_spec((DH, D)),
            _full_spec((1, D)), _full_spec((D, D)),
        ],
        out_specs=_row_spec(BM),
        out_shape=jax.ShapeDtypeStruct((NPAD, D), _f32),
    )(a1L, a1R, d2, xpad, W1lT[:DH], W1lT[DH:], b1l.reshape(1, D), W1r.T)

    a2L, a2R = _agg_call(z1[:, :DH], z1[:, DH:], src, dst2)

    P, Q = pl.pallas_call(
        _lin2_body,
        grid=(NPAD // BM,),
        in_specs=[
            pl.BlockSpec((BM, DH), lambda i: (i, 0)),
            pl.BlockSpec((BM, DH), lambda i: (i, 0)),
            pl.BlockSpec((BM, 1), lambda i: (i, 0)),
            _row_spec(BM),
            _full_spec((DH, D)), _full_spec((DH, D)),
            _full_spec((1, D)), _full_spec((D, D)),
            _full_spec((D, D)), _full_spec((1, D)), _full_spec((D, D)),
        ],
        out_specs=[_row_spec(BM), _row_spec(BM)],
        out_shape=[jax.ShapeDtypeStruct((NPAD, D), _f32),
                   jax.ShapeDtypeStruct((NPAD, D), _f32)],
    )(a2L, a2R, d2, z1, W2lT[:DH], W2lT[DH:], b2l.reshape(1, D), W2r.T,
      Wa[:, :D].T, ba.reshape(1, D), Wa[:, D:].T)

    R = _dec_call(P, Q, se, de)

    preds = pl.pallas_call(
        _dec2_body,
        grid=(PEPAD // BM3,),
        in_specs=[
            _row_spec(BM3),
            _full_spec((1, D)),
            pl.BlockSpec((1, 1), lambda i: (0, 0)),
        ],
        out_specs=pl.BlockSpec((BM3, 1), lambda i: (i, 0)),
        out_shape=jax.ShapeDtypeStruct((PEPAD, 1), _f32),
    )(R, Wb, bb.reshape(1, 1))

    preds = preds[:, 0]
    return (preds[:PE], preds[PE:PE2])
